# baseline (speedup 1.0000x reference)
import jax
import jax.numpy as jnp
from jax import lax
from jax.experimental import pallas as pl
from jax.experimental.pallas import tpu as pltpu
from jax.experimental.pallas import tpu_sc as plsc

N_NODES = 10000
N_EDGES = 320000
D = 128

NC = 2
NS = 16
NW = NC * NS
K = 128
ER = 2560
GRW = ER // NW
GB = 8
GG = GRW // GB
NP = 10240
RPT = NP // NS
RCH = 128


def _sc_scatter_body(nfeat_hbm, src_hbm, dst_hbm,
                     acc_out, deg_out,
                     srcba, dstba, srcbb, dstbb, rows0, rows1,
                     deg_io, ones_v,
                     semg0, semg1, semis, semid,
                     acc_sh, deg_sh):
    c = lax.axis_index("c")
    s = lax.axis_index("s")
    wid = c * NS + s
    gbase = wid * GRW

    rows = (rows0, rows1)
    semg = (semg0, semg1)

    one16 = jnp.full((16,), 1.0, dtype=jnp.float32)
    zero16 = jnp.zeros((16,), dtype=jnp.float32)
    for i in range(K // 16):
        ones_v[pl.ds(i * 16, 16)] = one16
    for i in range(RCH // 16):
        deg_io[pl.ds(i * 16, 16)] = zero16

    for r in range(RPT // RCH):
        r0 = s * RPT + r * RCH
        pltpu.sync_copy(nfeat_hbm.at[pl.ds(r0, RCH)], rows0)
        pltpu.sync_copy(rows0, acc_sh.at[pl.ds(r0, RCH)])
        pltpu.sync_copy(deg_io, deg_sh.at[pl.ds(r0, RCH)])
    plsc.subcore_barrier()

    def idx_load(group, srcbx, dstbx, sync):
        r0 = pl.multiple_of(gbase + group * GB, 8)
        if sync:
            pltpu.sync_copy(src_hbm.at[pl.ds(r0, GB)], srcbx)
            pltpu.sync_copy(dst_hbm.at[pl.ds(r0, GB)], dstbx)
        else:
            pltpu.async_copy(src_hbm.at[pl.ds(r0, GB)], srcbx, semis)
            pltpu.async_copy(dst_hbm.at[pl.ds(r0, GB)], dstbx, semid)

    def idx_drain(srcbx, dstbx):
        pltpu.make_async_copy(src_hbm.at[pl.ds(0, GB)], srcbx, semis).wait()
        pltpu.make_async_copy(dst_hbm.at[pl.ds(0, GB)], dstbx, semid).wait()

    def gather(srcbx, j, p):
        pltpu.async_copy(nfeat_hbm.at[srcbx.at[j]], rows[p], semg[p])

    def gather_wait(p):
        pltpu.make_async_copy(nfeat_hbm.at[srcba.at[0]], rows[p],
                              semg[p]).wait()

    idx_load(0, srcba, dstba, True)
    gather(srcba, 0, 0)

    def pair(h, carry):
        idx_load(2 * h + 1, srcbb, dstbb, False)
        for j in range(2 * GB):
            p = j & 1
            in_a = j < GB
            dstbx = dstba if in_a else dstbb
            jj = j % GB
            gather_wait(p)
            if j == 2 * GB - 1:
                @pl.when(h < GG // 2 - 1)
                def _():
                    idx_drain(srcba, dstba)
                    gather(srcba, 0, 1 - p)
            else:
                nj = j + 1
                if nj == GB:
                    idx_drain(srcbb, dstbb)
                gather(srcba if nj < GB else srcbb, nj % GB, 1 - p)
            pltpu.sync_copy(rows[p], acc_sh.at[dstbx.at[jj]], add=True)
            pltpu.sync_copy(ones_v, deg_sh.at[dstbx.at[jj]], add=True)
            if j == GB - 1:
                @pl.when(h < GG // 2 - 1)
                def _():
                    idx_load(2 * h + 2, srcba, dstba, False)
        return carry

    lax.fori_loop(0, GG // 2, pair, 0)
    plsc.subcore_barrier()

    for r in range(RPT // RCH):
        r0 = s * RPT + r * RCH
        pltpu.sync_copy(acc_sh.at[pl.ds(r0, RCH)], rows0)
        pltpu.sync_copy(rows0, acc_out.at[c].at[pl.ds(r0, RCH)])
        pltpu.sync_copy(deg_sh.at[pl.ds(r0, RCH)], deg_io)
        pltpu.sync_copy(deg_io, deg_out.at[c].at[pl.ds(r0, RCH)])


_sc_scatter = pl.kernel(
    _sc_scatter_body,
    out_type=[
        jax.ShapeDtypeStruct((NC, NP, D), jnp.float32),
        jax.ShapeDtypeStruct((NC, NP), jnp.float32),
    ],
    mesh=plsc.VectorSubcoreMesh(core_axis_name="c", subcore_axis_name="s",
                                num_cores=NC, num_subcores=NS),
    scratch_types=[
        pltpu.VMEM((GB, K), jnp.int32),
        pltpu.VMEM((GB, K), jnp.int32),
        pltpu.VMEM((GB, K), jnp.int32),
        pltpu.VMEM((GB, K), jnp.int32),
        pltpu.VMEM((K, D), jnp.float32),
        pltpu.VMEM((K, D), jnp.float32),
        pltpu.VMEM((RCH,), jnp.float32),
        pltpu.VMEM((K,), jnp.float32),
        pltpu.SemaphoreType.DMA,
        pltpu.SemaphoreType.DMA,
        pltpu.SemaphoreType.DMA,
        pltpu.SemaphoreType.DMA,
        pltpu.VMEM_SHARED((NP, D), jnp.float32),
        pltpu.VMEM_SHARED((NP,), jnp.float32),
    ],
)


def _tc_combine_body(acc_ref, deg_ref, wt_ref, b_ref, out_ref):
    a = acc_ref[0] + acc_ref[1]
    d = deg_ref[0] + deg_ref[1] + 1.0
    y = jnp.dot(a, wt_ref[...], preferred_element_type=jnp.float32)
    out_ref[...] = (y + b_ref[...]) / d + b_ref[...]


def _tc_combine(acc, deg, wt, b):
    blk = 1000
    grid = (N_NODES // blk,)
    return pl.pallas_call(
        _tc_combine_body,
        grid=grid,
        in_specs=[
            pl.BlockSpec((NC, blk, D), lambda i: (0, i, 0)),
            pl.BlockSpec((NC, blk, 1), lambda i: (0, i, 0)),
            pl.BlockSpec((D, D), lambda i: (0, 0)),
            pl.BlockSpec((1, D), lambda i: (0, 0)),
        ],
        out_specs=pl.BlockSpec((blk, D), lambda i: (i, 0)),
        out_shape=jax.ShapeDtypeStruct((N_NODES, D), jnp.float32),
    )(acc, deg, wt, b)


def kernel(nfeat, edge_index, W_neigh, b_neigh):
    src = edge_index[0].astype(jnp.int32)
    dst = edge_index[1].astype(jnp.int32)
    npad = ER * K - N_EDGES
    src2 = jnp.concatenate(
        [src, jnp.full((npad,), N_NODES, jnp.int32)]).reshape(ER, K)
    dst2 = jnp.concatenate(
        [dst, jnp.full((npad,), N_NODES, jnp.int32)]).reshape(ER, K)
    nfeat_p = jnp.pad(nfeat, ((0, NP - N_NODES), (0, 0)))
    acc, degf = _sc_scatter(nfeat_p, src2, dst2)
    acc = acc[:, :N_NODES]
    deg = degf[:, :N_NODES, None]
    return _tc_combine(acc, deg, W_neigh.T, b_neigh[None, :])


# speedup vs baseline: 1.1894x; 1.1894x over previous
import jax
import jax.numpy as jnp
from jax import lax
from jax.experimental import pallas as pl
from jax.experimental.pallas import tpu as pltpu
from jax.experimental.pallas import tpu_sc as plsc

N_NODES = 10000
N_EDGES = 320000
D = 128

NC = 2
NS = 16
NW = NC * NS
K = 128
ER = 2560
GRW = ER // NW
GB = 8
GG = GRW // GB
NP = 10240
RPT = NP // NS
RCH = 128


def _sc_scatter_body(nfeat_hbm, src_hbm, dst_hbm,
                     acc_out, deg_out,
                     srcba, dstba, srcbb, dstbb, rows0, rows1,
                     deg_io, ones_v,
                     semg0, semg1, semis, semid,
                     acc_sh, deg_sh):
    c = lax.axis_index("c")
    s = lax.axis_index("s")
    wid = c * NS + s
    gbase = wid * GRW

    rows = (rows0, rows1)
    semg = (semg0, semg1)

    one16 = jnp.full((16,), 1.0, dtype=jnp.float32)
    zero16 = jnp.zeros((16,), dtype=jnp.float32)
    for i in range(K // 16):
        ones_v[pl.ds(i * 16, 16)] = one16
    for i in range(RCH // 16):
        deg_io[pl.ds(i * 16, 16)] = zero16

    for r in range(RPT // RCH):
        r0 = s * RPT + r * RCH
        pltpu.sync_copy(nfeat_hbm.at[pl.ds(r0, RCH)], rows0)
        pltpu.sync_copy(rows0, acc_sh.at[pl.ds(r0, RCH)])
        pltpu.sync_copy(deg_io, deg_sh.at[pl.ds(r0, RCH)])
    plsc.subcore_barrier()

    ebase = wid * GRW * K

    def issue(i, idx_sx, idx_dx, rows_x, sem_x):
        b = pl.multiple_of(ebase + i * K, 8)
        pltpu.sync_copy(src_hbm.at[pl.ds(b, K)], idx_sx)
        pltpu.sync_copy(dst_hbm.at[pl.ds(b, K)], idx_dx)
        pltpu.async_copy(nfeat_hbm.at[idx_sx], rows_x, sem_x)

    def drain(idx_dx, rows_x, sem_x):
        pltpu.make_async_copy(nfeat_hbm.at[srcba], rows_x, sem_x).wait()
        pltpu.sync_copy(rows_x, acc_sh.at[idx_dx], add=True)
        pltpu.sync_copy(ones_v, deg_sh.at[idx_dx], add=True)

    issue(0, srcba, dstba, rows0, semg0)

    def pair(g, carry):
        issue(2 * g + 1, srcbb, dstbb, rows1, semg1)
        drain(dstba, rows0, semg0)

        @pl.when(2 * g + 2 < GRW)
        def _():
            issue(2 * g + 2, srcba, dstba, rows0, semg0)

        drain(dstbb, rows1, semg1)
        return carry

    lax.fori_loop(0, GRW // 2, pair, 0)
    plsc.subcore_barrier()

    for r in range(RPT // RCH):
        r0 = s * RPT + r * RCH
        pltpu.sync_copy(acc_sh.at[pl.ds(r0, RCH)], rows0)
        pltpu.sync_copy(rows0, acc_out.at[c].at[pl.ds(r0, RCH)])
        pltpu.sync_copy(deg_sh.at[pl.ds(r0, RCH)], deg_io)
        pltpu.sync_copy(deg_io, deg_out.at[c].at[pl.ds(r0, RCH)])


_sc_scatter = pl.kernel(
    _sc_scatter_body,
    out_type=[
        jax.ShapeDtypeStruct((NC, NP, D), jnp.float32),
        jax.ShapeDtypeStruct((NC, NP), jnp.float32),
    ],
    mesh=plsc.VectorSubcoreMesh(core_axis_name="c", subcore_axis_name="s",
                                num_cores=NC, num_subcores=NS),
    scratch_types=[
        pltpu.VMEM((K,), jnp.int32),
        pltpu.VMEM((K,), jnp.int32),
        pltpu.VMEM((K,), jnp.int32),
        pltpu.VMEM((K,), jnp.int32),
        pltpu.VMEM((K, D), jnp.float32),
        pltpu.VMEM((K, D), jnp.float32),
        pltpu.VMEM((RCH,), jnp.float32),
        pltpu.VMEM((K,), jnp.float32),
        pltpu.SemaphoreType.DMA,
        pltpu.SemaphoreType.DMA,
        pltpu.SemaphoreType.DMA,
        pltpu.SemaphoreType.DMA,
        pltpu.VMEM_SHARED((NP, D), jnp.float32),
        pltpu.VMEM_SHARED((NP,), jnp.float32),
    ],
)


def _tc_combine_body(acc_ref, deg_ref, wt_ref, b_ref, out_ref):
    a = acc_ref[0] + acc_ref[1]
    d = deg_ref[0] + deg_ref[1] + 1.0
    y = jnp.dot(a, wt_ref[...], preferred_element_type=jnp.float32)
    out_ref[...] = (y + b_ref[...]) / d + b_ref[...]


def _tc_combine(acc, deg, wt, b):
    blk = 1000
    grid = (N_NODES // blk,)
    return pl.pallas_call(
        _tc_combine_body,
        grid=grid,
        in_specs=[
            pl.BlockSpec((NC, blk, D), lambda i: (0, i, 0)),
            pl.BlockSpec((NC, blk, 1), lambda i: (0, i, 0)),
            pl.BlockSpec((D, D), lambda i: (0, 0)),
            pl.BlockSpec((1, D), lambda i: (0, 0)),
        ],
        out_specs=pl.BlockSpec((blk, D), lambda i: (i, 0)),
        out_shape=jax.ShapeDtypeStruct((N_NODES, D), jnp.float32),
    )(acc, deg, wt, b)


def kernel(nfeat, edge_index, W_neigh, b_neigh):
    src = edge_index[0].astype(jnp.int32)
    dst = edge_index[1].astype(jnp.int32)
    npad = ER * K - N_EDGES
    src2 = jnp.concatenate([src, jnp.full((npad,), N_NODES, jnp.int32)])
    dst2 = jnp.concatenate([dst, jnp.full((npad,), N_NODES, jnp.int32)])
    nfeat_p = jnp.pad(nfeat, ((0, NP - N_NODES), (0, 0)))
    acc, degf = _sc_scatter(nfeat_p, src2, dst2)
    acc = acc[:, :N_NODES]
    deg = degf[:, :N_NODES, None]
    return _tc_combine(acc, deg, W_neigh.T, b_neigh[None, :])


# speedup vs baseline: 1.6882x; 1.4193x over previous
import jax
import jax.numpy as jnp
from jax import lax
from jax.experimental import pallas as pl
from jax.experimental.pallas import tpu as pltpu
from jax.experimental.pallas import tpu_sc as plsc

N_NODES = 10000
N_EDGES = 320000
D = 128

NC = 2
NS = 16
NW = NC * NS
K = 64
GRW = -(-N_EDGES // (NW * K))
EP = NW * GRW * K
NP = 10240
RPT = NP // NS
RCH = 128


def _sc_scatter_body(nfeat_hbm, src_hbm, dst_hbm,
                     acc_out, deg_out,
                     srcba, dstba, srcbb, dstbb, rows0, rows1, rows_io,
                     deg_io, ones_v,
                     semg0, semg1, semis, semid,
                     acc_sh, deg_sh):
    c = lax.axis_index("c")
    s = lax.axis_index("s")
    wid = c * NS + s
    gbase = wid * GRW

    rows = (rows0, rows1)
    semg = (semg0, semg1)

    one16 = jnp.full((16,), 1.0, dtype=jnp.float32)
    zero16 = jnp.zeros((16,), dtype=jnp.float32)
    for i in range(K // 16):
        ones_v[pl.ds(i * 16, 16)] = one16
    for i in range(RCH // 16):
        deg_io[pl.ds(i * 16, 16)] = zero16

    for r in range(RPT // RCH):
        r0 = s * RPT + r * RCH
        pltpu.sync_copy(nfeat_hbm.at[pl.ds(r0, RCH)], rows_io)
        pltpu.sync_copy(rows_io, acc_sh.at[pl.ds(r0, RCH)])
        pltpu.sync_copy(deg_io, deg_sh.at[pl.ds(r0, RCH)])
    plsc.subcore_barrier()

    ebase = wid * GRW * K

    def issue(i, idx_sx, idx_dx, rows_x, sem_x):
        b = pl.multiple_of(ebase + i * K, 8)
        pltpu.sync_copy(src_hbm.at[pl.ds(b, K)], idx_sx)
        pltpu.sync_copy(dst_hbm.at[pl.ds(b, K)], idx_dx)
        pltpu.async_copy(nfeat_hbm.at[idx_sx], rows_x, sem_x)

    def drain(idx_dx, rows_x, sem_x):
        pltpu.make_async_copy(nfeat_hbm.at[srcba], rows_x, sem_x).wait()
        pltpu.sync_copy(rows_x, acc_sh.at[idx_dx], add=True)
        pltpu.sync_copy(ones_v, deg_sh.at[idx_dx], add=True)

    issue(0, srcba, dstba, rows0, semg0)

    def pair(g, carry):
        issue(2 * g + 1, srcbb, dstbb, rows1, semg1)
        drain(dstba, rows0, semg0)

        @pl.when(2 * g + 2 < GRW)
        def _():
            issue(2 * g + 2, srcba, dstba, rows0, semg0)

        drain(dstbb, rows1, semg1)
        return carry

    lax.fori_loop(0, GRW // 2, pair, 0)
    if GRW % 2 == 1:
        drain(dstba, rows0, semg0)
    plsc.subcore_barrier()

    for r in range(RPT // RCH):
        r0 = s * RPT + r * RCH
        pltpu.sync_copy(acc_sh.at[pl.ds(r0, RCH)], rows_io)
        pltpu.sync_copy(rows_io, acc_out.at[c].at[pl.ds(r0, RCH)])
        pltpu.sync_copy(deg_sh.at[pl.ds(r0, RCH)], deg_io)
        pltpu.sync_copy(deg_io, deg_out.at[c].at[pl.ds(r0, RCH)])


_sc_scatter = pl.kernel(
    _sc_scatter_body,
    out_type=[
        jax.ShapeDtypeStruct((NC, NP, D), jnp.float32),
        jax.ShapeDtypeStruct((NC, NP), jnp.float32),
    ],
    mesh=plsc.VectorSubcoreMesh(core_axis_name="c", subcore_axis_name="s",
                                num_cores=NC, num_subcores=NS),
    scratch_types=[
        pltpu.VMEM((K,), jnp.int32),
        pltpu.VMEM((K,), jnp.int32),
        pltpu.VMEM((K,), jnp.int32),
        pltpu.VMEM((K,), jnp.int32),
        pltpu.VMEM((K, D), jnp.float32),
        pltpu.VMEM((K, D), jnp.float32),
        pltpu.VMEM((RCH, D), jnp.float32),
        pltpu.VMEM((RCH,), jnp.float32),
        pltpu.VMEM((K,), jnp.float32),
        pltpu.SemaphoreType.DMA,
        pltpu.SemaphoreType.DMA,
        pltpu.SemaphoreType.DMA,
        pltpu.SemaphoreType.DMA,
        pltpu.VMEM_SHARED((NP, D), jnp.float32),
        pltpu.VMEM_SHARED((NP,), jnp.float32),
    ],
)


def _tc_combine_body(acc_ref, deg_ref, wt_ref, b_ref, out_ref):
    a = acc_ref[0] + acc_ref[1]
    d = deg_ref[0] + deg_ref[1] + 1.0
    y = jnp.dot(a, wt_ref[...], preferred_element_type=jnp.float32)
    out_ref[...] = (y + b_ref[...]) / d + b_ref[...]


def _tc_combine(acc, deg, wt, b):
    blk = 1000
    grid = (N_NODES // blk,)
    return pl.pallas_call(
        _tc_combine_body,
        grid=grid,
        in_specs=[
            pl.BlockSpec((NC, blk, D), lambda i: (0, i, 0)),
            pl.BlockSpec((NC, blk, 1), lambda i: (0, i, 0)),
            pl.BlockSpec((D, D), lambda i: (0, 0)),
            pl.BlockSpec((1, D), lambda i: (0, 0)),
        ],
        out_specs=pl.BlockSpec((blk, D), lambda i: (i, 0)),
        out_shape=jax.ShapeDtypeStruct((N_NODES, D), jnp.float32),
    )(acc, deg, wt, b)


def kernel(nfeat, edge_index, W_neigh, b_neigh):
    src = edge_index[0].astype(jnp.int32)
    dst = edge_index[1].astype(jnp.int32)
    npad = EP - N_EDGES
    src2 = jnp.concatenate([src, jnp.full((npad,), N_NODES, jnp.int32)])
    dst2 = jnp.concatenate([dst, jnp.full((npad,), N_NODES, jnp.int32)])
    nfeat_p = jnp.pad(nfeat, ((0, NP - N_NODES), (0, 0)))
    acc, degf = _sc_scatter(nfeat_p, src2, dst2)
    acc = acc[:, :N_NODES]
    deg = degf[:, :N_NODES, None]
    return _tc_combine(acc, deg, W_neigh.T, b_neigh[None, :])


# speedup vs baseline: 1.7104x; 1.0131x over previous
import jax
import jax.numpy as jnp
from jax import lax
from jax.experimental import pallas as pl
from jax.experimental.pallas import tpu as pltpu
from jax.experimental.pallas import tpu_sc as plsc

N_NODES = 10000
N_EDGES = 320000
D = 128

NC = 2
NS = 16
NW = NC * NS
K = 96
GRW = -(-N_EDGES // (NW * K))
EP = NW * GRW * K
NP = 10240
RPT = NP // NS
RCH = 128


def _sc_scatter_body(nfeat_hbm, src_hbm, dst_hbm,
                     acc_out, deg_out,
                     srcba, dstba, srcbb, dstbb, rows0, rows1, rows_io,
                     deg_io, ones_v,
                     semg0, semg1, semis, semid,
                     acc_sh, deg_sh):
    c = lax.axis_index("c")
    s = lax.axis_index("s")
    wid = c * NS + s
    gbase = wid * GRW

    rows = (rows0, rows1)
    semg = (semg0, semg1)

    one16 = jnp.full((16,), 1.0, dtype=jnp.float32)
    zero16 = jnp.zeros((16,), dtype=jnp.float32)
    for i in range(K // 16):
        ones_v[pl.ds(i * 16, 16)] = one16
    for i in range(RCH // 16):
        deg_io[pl.ds(i * 16, 16)] = zero16

    for r in range(RPT // RCH):
        r0 = s * RPT + r * RCH
        pltpu.sync_copy(nfeat_hbm.at[pl.ds(r0, RCH)], rows_io)
        pltpu.sync_copy(rows_io, acc_sh.at[pl.ds(r0, RCH)])
        pltpu.sync_copy(deg_io, deg_sh.at[pl.ds(r0, RCH)])
    plsc.subcore_barrier()

    ebase = wid * GRW * K

    def issue(i, idx_sx, idx_dx, rows_x, sem_x):
        b = pl.multiple_of(ebase + i * K, 8)
        pltpu.sync_copy(src_hbm.at[pl.ds(b, K)], idx_sx)
        pltpu.sync_copy(dst_hbm.at[pl.ds(b, K)], idx_dx)
        pltpu.async_copy(nfeat_hbm.at[idx_sx], rows_x, sem_x)

    def drain(idx_dx, rows_x, sem_x):
        pltpu.make_async_copy(nfeat_hbm.at[srcba], rows_x, sem_x).wait()
        pltpu.sync_copy(rows_x, acc_sh.at[idx_dx], add=True)
        pltpu.sync_copy(ones_v, deg_sh.at[idx_dx], add=True)

    issue(0, srcba, dstba, rows0, semg0)

    def pair(g, carry):
        issue(2 * g + 1, srcbb, dstbb, rows1, semg1)
        drain(dstba, rows0, semg0)

        @pl.when(2 * g + 2 < GRW)
        def _():
            issue(2 * g + 2, srcba, dstba, rows0, semg0)

        drain(dstbb, rows1, semg1)
        return carry

    lax.fori_loop(0, GRW // 2, pair, 0)
    if GRW % 2 == 1:
        drain(dstba, rows0, semg0)
    plsc.subcore_barrier()

    for r in range(RPT // RCH):
        r0 = s * RPT + r * RCH
        pltpu.sync_copy(acc_sh.at[pl.ds(r0, RCH)], rows_io)
        pltpu.sync_copy(rows_io, acc_out.at[c].at[pl.ds(r0, RCH)])
        pltpu.sync_copy(deg_sh.at[pl.ds(r0, RCH)], deg_io)
        pltpu.sync_copy(deg_io, deg_out.at[c].at[pl.ds(r0, RCH)])


_sc_scatter = pl.kernel(
    _sc_scatter_body,
    out_type=[
        jax.ShapeDtypeStruct((NC, NP, D), jnp.float32),
        jax.ShapeDtypeStruct((NC, NP), jnp.float32),
    ],
    mesh=plsc.VectorSubcoreMesh(core_axis_name="c", subcore_axis_name="s",
                                num_cores=NC, num_subcores=NS),
    scratch_types=[
        pltpu.VMEM((K,), jnp.int32),
        pltpu.VMEM((K,), jnp.int32),
        pltpu.VMEM((K,), jnp.int32),
        pltpu.VMEM((K,), jnp.int32),
        pltpu.VMEM((K, D), jnp.float32),
        pltpu.VMEM((K, D), jnp.float32),
        pltpu.VMEM((RCH, D), jnp.float32),
        pltpu.VMEM((RCH,), jnp.float32),
        pltpu.VMEM((K,), jnp.float32),
        pltpu.SemaphoreType.DMA,
        pltpu.SemaphoreType.DMA,
        pltpu.SemaphoreType.DMA,
        pltpu.SemaphoreType.DMA,
        pltpu.VMEM_SHARED((NP, D), jnp.float32),
        pltpu.VMEM_SHARED((NP,), jnp.float32),
    ],
)


def _tc_combine_body(acc_ref, deg_ref, wt_ref, b_ref, out_ref):
    a = acc_ref[0] + acc_ref[1]
    d = deg_ref[0] + deg_ref[1] + 1.0
    y = jnp.dot(a, wt_ref[...], preferred_element_type=jnp.float32)
    out_ref[...] = (y + b_ref[...]) / d + b_ref[...]


def _tc_combine(acc, deg, wt, b):
    blk = 1000
    grid = (N_NODES // blk,)
    return pl.pallas_call(
        _tc_combine_body,
        grid=grid,
        in_specs=[
            pl.BlockSpec((NC, blk, D), lambda i: (0, i, 0)),
            pl.BlockSpec((NC, blk, 1), lambda i: (0, i, 0)),
            pl.BlockSpec((D, D), lambda i: (0, 0)),
            pl.BlockSpec((1, D), lambda i: (0, 0)),
        ],
        out_specs=pl.BlockSpec((blk, D), lambda i: (i, 0)),
        out_shape=jax.ShapeDtypeStruct((N_NODES, D), jnp.float32),
    )(acc, deg, wt, b)


def kernel(nfeat, edge_index, W_neigh, b_neigh):
    src = edge_index[0].astype(jnp.int32)
    dst = edge_index[1].astype(jnp.int32)
    npad = EP - N_EDGES
    src2 = jnp.concatenate([src, jnp.full((npad,), N_NODES, jnp.int32)])
    dst2 = jnp.concatenate([dst, jnp.full((npad,), N_NODES, jnp.int32)])
    nfeat_p = jnp.pad(nfeat, ((0, NP - N_NODES), (0, 0)))
    acc, degf = _sc_scatter(nfeat_p, src2, dst2)
    acc = acc[:, :N_NODES]
    deg = degf[:, :N_NODES, None]
    return _tc_combine(acc, deg, W_neigh.T, b_neigh[None, :])


# speedup vs baseline: 2.1166x; 1.2375x over previous
import jax
import jax.numpy as jnp
from jax import lax
from jax.experimental import pallas as pl
from jax.experimental.pallas import tpu as pltpu
from jax.experimental.pallas import tpu_sc as plsc

N_NODES = 10000
N_EDGES = 320000
D = 128

NC = 2
NS = 16
NW = NC * NS
K = 80
GRW = -(-N_EDGES // (NW * K))
EP = NW * GRW * K
NP = 10240
RPT = NP // NS
RCH = 128


def _sc_scatter_body(nfeat_hbm, src_hbm, dst_hbm,
                     acc_out, deg_out,
                     srcba, dstba, srcbb, dstbb, rows0, rows1, rows_io,
                     deg_io, ones_v,
                     semg0, semg1, semis, semid,
                     acc_sh, deg_sh):
    c = lax.axis_index("c")
    s = lax.axis_index("s")
    wid = c * NS + s
    gbase = wid * GRW

    rows = (rows0, rows1)
    semg = (semg0, semg1)

    one16 = jnp.full((16,), 1.0, dtype=jnp.float32)
    zero16 = jnp.zeros((16,), dtype=jnp.float32)
    for i in range(K // 16):
        ones_v[pl.ds(i * 16, 16)] = one16
    for i in range(RCH // 16):
        deg_io[pl.ds(i * 16, 16)] = zero16

    for r in range(RPT // RCH):
        r0 = s * RPT + r * RCH
        pltpu.sync_copy(nfeat_hbm.at[pl.ds(r0, RCH)], rows_io)
        pltpu.sync_copy(rows_io, acc_sh.at[pl.ds(r0, RCH)])
        pltpu.sync_copy(deg_io, deg_sh.at[pl.ds(r0, RCH)])
    plsc.subcore_barrier()

    ebase = wid * GRW * K

    def issue(i, idx_sx, idx_dx, rows_x, sem_x):
        b = pl.multiple_of(ebase + i * K, 8)
        pltpu.sync_copy(src_hbm.at[pl.ds(b, K)], idx_sx)
        pltpu.sync_copy(dst_hbm.at[pl.ds(b, K)], idx_dx)
        pltpu.async_copy(nfeat_hbm.at[idx_sx], rows_x, sem_x)

    def drain(idx_dx, rows_x, sem_x):
        pltpu.make_async_copy(nfeat_hbm.at[srcba], rows_x, sem_x).wait()
        pltpu.sync_copy(rows_x, acc_sh.at[idx_dx], add=True)
        pltpu.sync_copy(ones_v, deg_sh.at[idx_dx], add=True)

    issue(0, srcba, dstba, rows0, semg0)

    def pair(g, carry):
        issue(2 * g + 1, srcbb, dstbb, rows1, semg1)
        drain(dstba, rows0, semg0)

        @pl.when(2 * g + 2 < GRW)
        def _():
            issue(2 * g + 2, srcba, dstba, rows0, semg0)

        drain(dstbb, rows1, semg1)
        return carry

    lax.fori_loop(0, GRW // 2, pair, 0)
    if GRW % 2 == 1:
        drain(dstba, rows0, semg0)
    plsc.subcore_barrier()

    for r in range(RPT // RCH):
        r0 = s * RPT + r * RCH
        pltpu.sync_copy(acc_sh.at[pl.ds(r0, RCH)], rows_io)
        pltpu.sync_copy(rows_io, acc_out.at[c].at[pl.ds(r0, RCH)])
        pltpu.sync_copy(deg_sh.at[pl.ds(r0, RCH)], deg_io)
        pltpu.sync_copy(deg_io, deg_out.at[c].at[pl.ds(r0, RCH)])


_sc_scatter = pl.kernel(
    _sc_scatter_body,
    out_type=[
        jax.ShapeDtypeStruct((NC, NP, D), jnp.float32),
        jax.ShapeDtypeStruct((NC, NP), jnp.float32),
    ],
    mesh=plsc.VectorSubcoreMesh(core_axis_name="c", subcore_axis_name="s",
                                num_cores=NC, num_subcores=NS),
    scratch_types=[
        pltpu.VMEM((K,), jnp.int32),
        pltpu.VMEM((K,), jnp.int32),
        pltpu.VMEM((K,), jnp.int32),
        pltpu.VMEM((K,), jnp.int32),
        pltpu.VMEM((K, D), jnp.float32),
        pltpu.VMEM((K, D), jnp.float32),
        pltpu.VMEM((RCH, D), jnp.float32),
        pltpu.VMEM((RCH,), jnp.float32),
        pltpu.VMEM((K,), jnp.float32),
        pltpu.SemaphoreType.DMA,
        pltpu.SemaphoreType.DMA,
        pltpu.SemaphoreType.DMA,
        pltpu.SemaphoreType.DMA,
        pltpu.VMEM_SHARED((NP, D), jnp.float32),
        pltpu.VMEM_SHARED((NP,), jnp.float32),
    ],
)


def _tc_combine_body(acc_ref, deg_ref, wt_ref, b_ref, out_ref):
    a = acc_ref[0] + acc_ref[1]
    d = deg_ref[0] + deg_ref[1] + 1.0
    y = jnp.dot(a, wt_ref[...], preferred_element_type=jnp.float32)
    out_ref[...] = (y + b_ref[...]) / d + b_ref[...]


def _tc_combine(acc, deg, wt, b):
    blk = 1000
    grid = (N_NODES // blk,)
    return pl.pallas_call(
        _tc_combine_body,
        grid=grid,
        in_specs=[
            pl.BlockSpec((NC, blk, D), lambda i: (0, i, 0)),
            pl.BlockSpec((NC, blk, 1), lambda i: (0, i, 0)),
            pl.BlockSpec((D, D), lambda i: (0, 0)),
            pl.BlockSpec((1, D), lambda i: (0, 0)),
        ],
        out_specs=pl.BlockSpec((blk, D), lambda i: (i, 0)),
        out_shape=jax.ShapeDtypeStruct((N_NODES, D), jnp.float32),
    )(acc, deg, wt, b)


def kernel(nfeat, edge_index, W_neigh, b_neigh):
    src = edge_index[0].astype(jnp.int32)
    dst = edge_index[1].astype(jnp.int32)
    npad = EP - N_EDGES
    src2 = jnp.concatenate([src, jnp.full((npad,), N_NODES, jnp.int32)])
    dst2 = jnp.concatenate([dst, jnp.full((npad,), N_NODES, jnp.int32)])
    nfeat_p = jnp.pad(nfeat, ((0, NP - N_NODES), (0, 0)))
    acc, degf = _sc_scatter(nfeat_p, src2, dst2)
    acc = acc[:, :N_NODES]
    deg = degf[:, :N_NODES, None]
    return _tc_combine(acc, deg, W_neigh.T, b_neigh[None, :])


# speedup vs baseline: 2.1663x; 1.0235x over previous
import jax
import jax.numpy as jnp
from jax import lax
from jax.experimental import pallas as pl
from jax.experimental.pallas import tpu as pltpu
from jax.experimental.pallas import tpu_sc as plsc

N_NODES = 10000
N_EDGES = 320000
D = 128

NC = 2
NS = 16
NW = NC * NS
K = 80
GRW = -(-N_EDGES // (NW * K))
EP = NW * GRW * K
NP = 10240
RPT = NP // NS
RCH = 128


def _sc_scatter_body(nfeat_hbm, src_hbm, dst_hbm,
                     acc_out, deg_out,
                     srcba, dstba, srcbb, dstbb, rows0, rows1, rows_io,
                     deg_io, ones_v,
                     semg0, semg1, semis, semid,
                     acc_sh, deg_sh):
    c = lax.axis_index("c")
    s = lax.axis_index("s")
    wid = c * NS + s
    gbase = wid * GRW

    rows = (rows0, rows1)
    semg = (semg0, semg1)

    one16 = jnp.full((16,), 1.0, dtype=jnp.float32)
    zero16 = jnp.zeros((16,), dtype=jnp.float32)
    for i in range(K // 16):
        ones_v[pl.ds(i * 16, 16)] = one16
    for i in range(RCH // 16):
        deg_io[pl.ds(i * 16, 16)] = zero16

    for r in range(RPT // RCH):
        r0 = s * RPT + r * RCH
        pltpu.sync_copy(nfeat_hbm.at[pl.ds(r0, RCH)], rows_io)
        pltpu.sync_copy(rows_io, acc_sh.at[pl.ds(r0, RCH)])
        pltpu.sync_copy(deg_io, deg_sh.at[pl.ds(r0, RCH)])
    plsc.subcore_barrier()

    ebase = wid * GRW * K

    def issue(i, idx_sx, idx_dx, rows_x, sem_x):
        b = pl.multiple_of(ebase + i * K, 8)
        pltpu.sync_copy(src_hbm.at[pl.ds(b, K)], idx_sx)
        pltpu.sync_copy(dst_hbm.at[pl.ds(b, K)], idx_dx)
        pltpu.async_copy(nfeat_hbm.at[idx_sx], rows_x, sem_x)

    def drain(idx_dx, rows_x, sem_x):
        pltpu.make_async_copy(nfeat_hbm.at[srcba], rows_x, sem_x).wait()
        pltpu.sync_copy(rows_x, acc_sh.at[idx_dx], add=True)
        pltpu.sync_copy(ones_v, deg_sh.at[idx_dx], add=True)

    issue(0, srcba, dstba, rows0, semg0)

    def pair(g, carry):
        issue(2 * g + 1, srcbb, dstbb, rows1, semg1)
        drain(dstba, rows0, semg0)

        @pl.when(2 * g + 2 < GRW)
        def _():
            issue(2 * g + 2, srcba, dstba, rows0, semg0)

        drain(dstbb, rows1, semg1)
        return carry

    lax.fori_loop(0, GRW // 2, pair, 0)
    if GRW % 2 == 1:
        drain(dstba, rows0, semg0)
    plsc.subcore_barrier()

    for r in range(RPT // RCH):
        r0 = s * RPT + r * RCH
        pltpu.sync_copy(acc_sh.at[pl.ds(r0, RCH)], rows_io)
        pltpu.sync_copy(rows_io, acc_out.at[c].at[pl.ds(r0, RCH)])
        pltpu.sync_copy(deg_sh.at[pl.ds(r0, RCH)], deg_io)
        pltpu.sync_copy(deg_io, deg_out.at[c].at[pl.ds(r0, RCH)])


_sc_scatter = pl.kernel(
    _sc_scatter_body,
    out_type=[
        jax.ShapeDtypeStruct((NC, NP, D), jnp.float32),
        jax.ShapeDtypeStruct((NC, NP), jnp.float32),
    ],
    mesh=plsc.VectorSubcoreMesh(core_axis_name="c", subcore_axis_name="s",
                                num_cores=NC, num_subcores=NS),
    scratch_types=[
        pltpu.VMEM((K,), jnp.int32),
        pltpu.VMEM((K,), jnp.int32),
        pltpu.VMEM((K,), jnp.int32),
        pltpu.VMEM((K,), jnp.int32),
        pltpu.VMEM((K, D), jnp.float32),
        pltpu.VMEM((K, D), jnp.float32),
        pltpu.VMEM((RCH, D), jnp.float32),
        pltpu.VMEM((RCH,), jnp.float32),
        pltpu.VMEM((K,), jnp.float32),
        pltpu.SemaphoreType.DMA,
        pltpu.SemaphoreType.DMA,
        pltpu.SemaphoreType.DMA,
        pltpu.SemaphoreType.DMA,
        pltpu.VMEM_SHARED((NP, D), jnp.float32),
        pltpu.VMEM_SHARED((NP,), jnp.float32),
    ],
)


def _tc_combine_body(acc_ref, deg_ref, wt_ref, b_ref, out_ref):
    a = acc_ref[0] + acc_ref[1]
    d = deg_ref[0] + deg_ref[1] + 1.0
    y = jnp.dot(a, wt_ref[...], preferred_element_type=jnp.float32)
    out_ref[...] = (y + b_ref[...]) / d + b_ref[...]


def _tc_combine(acc, deg, wt, b):
    blk = 1000
    grid = (N_NODES // blk,)
    return pl.pallas_call(
        _tc_combine_body,
        grid=grid,
        in_specs=[
            pl.BlockSpec((NC, blk, D), lambda i: (0, i, 0)),
            pl.BlockSpec((NC, blk, 1), lambda i: (0, i, 0)),
            pl.BlockSpec((D, D), lambda i: (0, 0)),
            pl.BlockSpec((1, D), lambda i: (0, 0)),
        ],
        out_specs=pl.BlockSpec((blk, D), lambda i: (i, 0)),
        out_shape=jax.ShapeDtypeStruct((N_NODES, D), jnp.float32),
    )(acc, deg, wt, b)


def kernel(nfeat, edge_index, W_neigh, b_neigh):
    src = edge_index[0].astype(jnp.int32)
    dst = edge_index[1].astype(jnp.int32)
    npad = EP - N_EDGES
    src2 = jnp.concatenate([src, jnp.full((npad,), N_NODES, jnp.int32)])
    dst2 = jnp.concatenate([dst, jnp.full((npad,), N_NODES, jnp.int32)])
    nfeat_p = jnp.pad(nfeat, ((0, NP - N_NODES), (0, 0)))
    acc, degf = _sc_scatter(nfeat_p, src2, dst2)
    return _tc_combine(acc, degf[:, :, None], W_neigh.T, b_neigh[None, :])


# speedup vs baseline: 2.4080x; 1.1116x over previous
import jax
import jax.numpy as jnp
from jax import lax
from jax.experimental import pallas as pl
from jax.experimental.pallas import tpu as pltpu
from jax.experimental.pallas import tpu_sc as plsc

N_NODES = 10000
N_EDGES = 320000
D = 128

NC = 2
NS = 16
NW = NC * NS
K = 80
GRW = -(-N_EDGES // (NW * K))
EP = NW * GRW * K
NP = 10240
RPT = NP // NS
RCH = 80
DCH = 128
NR = 3
NQ = 6
BODY = 6
T0 = 2
NLOOP = (GRW - T0 - 3) // BODY
TTAIL = T0 + NLOOP * BODY


def _sc_scatter_body(nfeat_hbm, edge_hbm,
                     acc_out, deg_out,
                     idx0, idx1, idx2, idx3, idx4, idx5,
                     rows0, rows1, rows2,
                     deg_io, ones_v,
                     sg0, sg1, sg2, ss0, ss1, ss2,
                     si0, si1, si2, si3, si4, si5,
                     acc_sh, deg_sh):
    c = lax.axis_index("c")
    s = lax.axis_index("s")
    wid = c * NS + s
    cbase = wid * GRW

    idxb = (idx0, idx1, idx2, idx3, idx4, idx5)
    rows = (rows0, rows1, rows2)
    semg = (sg0, sg1, sg2)
    semsc = (ss0, ss1, ss2)
    semi = (si0, si1, si2, si3, si4, si5)

    one16 = jnp.full((16,), 1.0, dtype=jnp.float32)
    zero16 = jnp.zeros((16,), dtype=jnp.float32)
    for i in range(K // 16):
        ones_v[pl.ds(i * 16, 16)] = one16
    for i in range(DCH // 16):
        deg_io[pl.ds(i * 16, 16)] = zero16

    for r in range(RPT // RCH):
        r0 = s * RPT + r * RCH
        pltpu.sync_copy(nfeat_hbm.at[pl.ds(r0, RCH)], rows0)
        pltpu.sync_copy(rows0, acc_sh.at[pl.ds(r0, RCH)])
    for r in range(RPT // DCH):
        r0 = s * RPT + r * DCH
        pltpu.sync_copy(deg_io, deg_sh.at[pl.ds(r0, DCH)])
    plsc.subcore_barrier()

    def idx_issue(t, q):
        pltpu.async_copy(edge_hbm.at[cbase + t], idxb[q], semi[q])

    def idx_drain(q):
        pltpu.make_async_copy(edge_hbm.at[cbase], idxb[q], semi[q]).wait()

    def gather_issue(t_q, t_r):
        pltpu.async_copy(nfeat_hbm.at[idxb[t_q].at[0]], rows[t_r],
                         semg[t_r])

    def gather_wait(t_r):
        pltpu.make_async_copy(nfeat_hbm.at[idxb[0].at[0]], rows[t_r],
                              semg[t_r]).wait()

    def scat_fire(t_q, t_r):
        pltpu.async_copy(rows[t_r], acc_sh.at[idxb[t_q].at[1]], semsc[t_r],
                         add=True)
        pltpu.async_copy(ones_v, deg_sh.at[idxb[t_q].at[1]], semsc[t_r],
                         add=True)

    def scat_drain(t_q, t_r):
        pltpu.make_async_copy(rows[t_r], acc_sh.at[idxb[t_q].at[1]],
                              semsc[t_r]).wait()
        pltpu.make_async_copy(ones_v, deg_sh.at[idxb[t_q].at[1]],
                              semsc[t_r]).wait()

    def step(t, tm, drain_sc, issue_idx, drain_idx, issue_g):
        rq, rr = tm % NQ, tm % NR
        if drain_sc:
            scat_drain((tm - 2) % NQ, (tm + 1) % NR)
        if issue_idx:
            idx_issue(t + 2, (tm + 2) % NQ)
        gather_wait(rr)
        scat_fire(rq, rr)
        if drain_idx:
            idx_drain((tm + 1) % NQ)
        if issue_g:
            gather_issue((tm + 1) % NQ, (tm + 1) % NR)

    pltpu.sync_copy(edge_hbm.at[cbase + 0], idx0)
    pltpu.sync_copy(edge_hbm.at[cbase + 1], idx1)
    gather_issue(0, 0)
    idx_issue(2, 2)
    gather_wait(0)
    scat_fire(0, 0)
    gather_issue(1, 1)
    idx_issue(3, 3)
    gather_wait(1)
    scat_fire(1, 1)
    idx_drain(2)
    gather_issue(2, 2)

    def body(h, carry):
        tb = T0 + h * BODY
        for j in range(BODY):
            step(tb + j, T0 + j, True, True, True, True)
        return carry

    lax.fori_loop(0, NLOOP, body, 0)

    for j in range(3):
        t = TTAIL + j
        step(t, t, True, t + 2 < GRW, t + 1 < GRW, t + 1 < GRW)
    scat_drain((GRW - 2) % NQ, (GRW - 2) % NR)
    scat_drain((GRW - 1) % NQ, (GRW - 1) % NR)
    plsc.subcore_barrier()

    for r in range(RPT // RCH):
        r0 = s * RPT + r * RCH
        pltpu.sync_copy(acc_sh.at[pl.ds(r0, RCH)], rows0)
        pltpu.sync_copy(rows0, acc_out.at[c].at[pl.ds(r0, RCH)])
    for r in range(RPT // DCH):
        r0 = s * RPT + r * DCH
        pltpu.sync_copy(deg_sh.at[pl.ds(r0, DCH)], deg_io)
        pltpu.sync_copy(deg_io, deg_out.at[c].at[pl.ds(r0, DCH)])


_sc_scatter = pl.kernel(
    _sc_scatter_body,
    out_type=[
        jax.ShapeDtypeStruct((NC, NP, D), jnp.float32),
        jax.ShapeDtypeStruct((NC, NP), jnp.float32),
    ],
    mesh=plsc.VectorSubcoreMesh(core_axis_name="c", subcore_axis_name="s",
                                num_cores=NC, num_subcores=NS),
    scratch_types=(
        [pltpu.VMEM((2, K), jnp.int32) for _ in range(NQ)]
        + [pltpu.VMEM((K, D), jnp.float32) for _ in range(NR)]
        + [
            pltpu.VMEM((DCH,), jnp.float32),
            pltpu.VMEM((K,), jnp.float32),
        ]
        + [pltpu.SemaphoreType.DMA for _ in range(NR + NR + NQ)]
        + [
            pltpu.VMEM_SHARED((NP, D), jnp.float32),
            pltpu.VMEM_SHARED((NP,), jnp.float32),
        ]
    ),
)


def _tc_combine_body(acc_ref, deg_ref, wt_ref, b_ref, out_ref):
    a = acc_ref[0] + acc_ref[1]
    d = deg_ref[0] + deg_ref[1] + 1.0
    y = jnp.dot(a, wt_ref[...], preferred_element_type=jnp.float32)
    out_ref[...] = (y + b_ref[...]) / d + b_ref[...]


def _tc_combine(acc, deg, wt, b):
    blk = 1000
    grid = (N_NODES // blk,)
    return pl.pallas_call(
        _tc_combine_body,
        grid=grid,
        in_specs=[
            pl.BlockSpec((NC, blk, D), lambda i: (0, i, 0)),
            pl.BlockSpec((NC, blk, 1), lambda i: (0, i, 0)),
            pl.BlockSpec((D, D), lambda i: (0, 0)),
            pl.BlockSpec((1, D), lambda i: (0, 0)),
        ],
        out_specs=pl.BlockSpec((blk, D), lambda i: (i, 0)),
        out_shape=jax.ShapeDtypeStruct((N_NODES, D), jnp.float32),
    )(acc, deg, wt, b)


def kernel(nfeat, edge_index, W_neigh, b_neigh):
    src = edge_index[0].astype(jnp.int32)
    dst = edge_index[1].astype(jnp.int32)
    npad = EP - N_EDGES
    src2 = jnp.concatenate([src, jnp.full((npad,), N_NODES, jnp.int32)])
    dst2 = jnp.concatenate([dst, jnp.full((npad,), N_NODES, jnp.int32)])
    edges = jnp.stack([src2.reshape(-1, K), dst2.reshape(-1, K)], axis=1)
    nfeat_p = jnp.pad(nfeat, ((0, NP - N_NODES), (0, 0)))
    acc, degf = _sc_scatter(nfeat_p, edges)
    return _tc_combine(acc, degf[:, :, None], W_neigh.T, b_neigh[None, :])


# speedup vs baseline: 2.4754x; 1.0280x over previous
import jax
import jax.numpy as jnp
from jax import lax
from jax.experimental import pallas as pl
from jax.experimental.pallas import tpu as pltpu
from jax.experimental.pallas import tpu_sc as plsc

N_NODES = 10000
N_EDGES = 320000
D = 128

NC = 2
NS = 16
NW = NC * NS
K = 80
GRW = -(-N_EDGES // (NW * K))
EP = NW * GRW * K
NP = 10240
RPT = NP // NS
RCH = 80
DCH = 128
NR = 3
NQ = 6
BODY = 6
T0 = 2
NLOOP = (GRW - T0 - 3) // BODY
TTAIL = T0 + NLOOP * BODY


def _sc_scatter_body(nfeat_hbm, edge_hbm,
                     acc_out, deg_out,
                     idx0, idx1, idx2, idx3, idx4, idx5,
                     rows0, rows1, rows2,
                     deg_io, ones_v,
                     sg0, sg1, sg2, ss0, ss1, ss2,
                     si0, si1, si2, si3, si4, si5,
                     acc_sh, deg_sh):
    c = lax.axis_index("c")
    s = lax.axis_index("s")
    wid = c * NS + s
    cbase = wid * GRW

    idxb = (idx0, idx1, idx2, idx3, idx4, idx5)
    rows = (rows0, rows1, rows2)
    semg = (sg0, sg1, sg2)
    semsc = (ss0, ss1, ss2)
    semi = (si0, si1, si2, si3, si4, si5)

    one16 = jnp.full((16,), 1.0, dtype=jnp.float32)
    zero16 = jnp.zeros((16,), dtype=jnp.float32)
    for i in range(K // 16):
        ones_v[pl.ds(i * 16, 16)] = one16
    for i in range(DCH // 16):
        deg_io[pl.ds(i * 16, 16)] = zero16

    r0 = s * RPT
    pltpu.sync_copy(nfeat_hbm.at[pl.ds(r0, RPT)], acc_sh.at[pl.ds(r0, RPT)])
    for r in range(RPT // DCH):
        r0 = s * RPT + r * DCH
        pltpu.sync_copy(deg_io, deg_sh.at[pl.ds(r0, DCH)])
    plsc.subcore_barrier()

    def idx_issue(t, q):
        pltpu.async_copy(edge_hbm.at[cbase + t], idxb[q], semi[q])

    def idx_drain(q):
        pltpu.make_async_copy(edge_hbm.at[cbase], idxb[q], semi[q]).wait()

    def gather_issue(t_q, t_r):
        pltpu.async_copy(nfeat_hbm.at[idxb[t_q].at[0]], rows[t_r],
                         semg[t_r])

    def gather_wait(t_r):
        pltpu.make_async_copy(nfeat_hbm.at[idxb[0].at[0]], rows[t_r],
                              semg[t_r]).wait()

    def scat_fire(t_q, t_r):
        pltpu.async_copy(rows[t_r], acc_sh.at[idxb[t_q].at[1]], semsc[t_r],
                         add=True)
        pltpu.async_copy(ones_v, deg_sh.at[idxb[t_q].at[1]], semsc[t_r],
                         add=True)

    def scat_drain(t_q, t_r):
        pltpu.make_async_copy(rows[t_r], acc_sh.at[idxb[t_q].at[1]],
                              semsc[t_r]).wait()
        pltpu.make_async_copy(ones_v, deg_sh.at[idxb[t_q].at[1]],
                              semsc[t_r]).wait()

    def step(t, tm, drain_sc, issue_idx, drain_idx, issue_g):
        rq, rr = tm % NQ, tm % NR
        if drain_sc:
            scat_drain((tm - 2) % NQ, (tm + 1) % NR)
        if issue_idx:
            idx_issue(t + 2, (tm + 2) % NQ)
        gather_wait(rr)
        scat_fire(rq, rr)
        if drain_idx:
            idx_drain((tm + 1) % NQ)
        if issue_g:
            gather_issue((tm + 1) % NQ, (tm + 1) % NR)

    pltpu.sync_copy(edge_hbm.at[cbase + 0], idx0)
    pltpu.sync_copy(edge_hbm.at[cbase + 1], idx1)
    gather_issue(0, 0)
    idx_issue(2, 2)
    gather_wait(0)
    scat_fire(0, 0)
    gather_issue(1, 1)
    idx_issue(3, 3)
    gather_wait(1)
    scat_fire(1, 1)
    idx_drain(2)
    gather_issue(2, 2)

    def body(h, carry):
        tb = T0 + h * BODY
        for j in range(BODY):
            step(tb + j, T0 + j, True, True, True, True)
        return carry

    lax.fori_loop(0, NLOOP, body, 0)

    for j in range(3):
        t = TTAIL + j
        step(t, t, True, t + 2 < GRW, t + 1 < GRW, t + 1 < GRW)
    scat_drain((GRW - 2) % NQ, (GRW - 2) % NR)
    scat_drain((GRW - 1) % NQ, (GRW - 1) % NR)
    plsc.subcore_barrier()

    r0 = s * RPT
    pltpu.sync_copy(acc_sh.at[pl.ds(r0, RPT)], acc_out.at[c].at[pl.ds(r0, RPT)])
    pltpu.sync_copy(deg_sh.at[pl.ds(r0, RPT)], deg_out.at[c].at[pl.ds(r0, RPT)])


_sc_scatter = pl.kernel(
    _sc_scatter_body,
    out_type=[
        jax.ShapeDtypeStruct((NC, NP, D), jnp.float32),
        jax.ShapeDtypeStruct((NC, NP), jnp.float32),
    ],
    mesh=plsc.VectorSubcoreMesh(core_axis_name="c", subcore_axis_name="s",
                                num_cores=NC, num_subcores=NS),
    scratch_types=(
        [pltpu.VMEM((2, K), jnp.int32) for _ in range(NQ)]
        + [pltpu.VMEM((K, D), jnp.float32) for _ in range(NR)]
        + [
            pltpu.VMEM((DCH,), jnp.float32),
            pltpu.VMEM((K,), jnp.float32),
        ]
        + [pltpu.SemaphoreType.DMA for _ in range(NR + NR + NQ)]
        + [
            pltpu.VMEM_SHARED((NP, D), jnp.float32),
            pltpu.VMEM_SHARED((NP,), jnp.float32),
        ]
    ),
)


def _tc_combine_body(acc_ref, deg_ref, wt_ref, b_ref, out_ref):
    a = acc_ref[0] + acc_ref[1]
    d = deg_ref[0] + deg_ref[1] + 1.0
    y = jnp.dot(a, wt_ref[...], preferred_element_type=jnp.float32)
    out_ref[...] = (y + b_ref[...]) / d + b_ref[...]


def _tc_combine(acc, deg, wt, b):
    blk = 1000
    grid = (N_NODES // blk,)
    return pl.pallas_call(
        _tc_combine_body,
        grid=grid,
        in_specs=[
            pl.BlockSpec((NC, blk, D), lambda i: (0, i, 0)),
            pl.BlockSpec((NC, blk, 1), lambda i: (0, i, 0)),
            pl.BlockSpec((D, D), lambda i: (0, 0)),
            pl.BlockSpec((1, D), lambda i: (0, 0)),
        ],
        out_specs=pl.BlockSpec((blk, D), lambda i: (i, 0)),
        out_shape=jax.ShapeDtypeStruct((N_NODES, D), jnp.float32),
    )(acc, deg, wt, b)


def kernel(nfeat, edge_index, W_neigh, b_neigh):
    src = edge_index[0].astype(jnp.int32)
    dst = edge_index[1].astype(jnp.int32)
    npad = EP - N_EDGES
    src2 = jnp.concatenate([src, jnp.full((npad,), N_NODES, jnp.int32)])
    dst2 = jnp.concatenate([dst, jnp.full((npad,), N_NODES, jnp.int32)])
    edges = jnp.stack([src2.reshape(-1, K), dst2.reshape(-1, K)], axis=1)
    nfeat_p = jnp.pad(nfeat, ((0, NP - N_NODES), (0, 0)))
    acc, degf = _sc_scatter(nfeat_p, edges)
    return _tc_combine(acc, degf[:, :, None], W_neigh.T, b_neigh[None, :])


# speedup vs baseline: 3.0351x; 1.2261x over previous
import jax
import jax.numpy as jnp
from jax import lax
from jax.experimental import pallas as pl
from jax.experimental.pallas import tpu as pltpu
from jax.experimental.pallas import tpu_sc as plsc

N_NODES = 10000
N_EDGES = 320000
D = 128

NC = 2
NS = 16
NW = NC * NS
K = 80
GRW = -(-N_EDGES // (NW * K))
EP = NW * GRW * K
NP = 10240
RPT = NP // NS
RCH = 80
DCH = 128
NR = 4
NQ = 6
BODY = 12
T0 = 2
NLOOP = (GRW - T0 - 4) // BODY
TTAIL = T0 + NLOOP * BODY


def _sc_scatter_body(nfeat_hbm, edge_hbm,
                     acc_out, deg_out,
                     idx0, idx1, idx2, idx3, idx4, idx5,
                     rows0, rows1, rows2, rows3,
                     deg_io, ones_v,
                     sg0, sg1, sg2, sg3, ss0, ss1, ss2, ss3,
                     si0, si1, si2, si3, si4, si5,
                     acc_sh, deg_sh):
    c = lax.axis_index("c")
    s = lax.axis_index("s")
    wid = c * NS + s
    cbase = wid * GRW

    idxb = (idx0, idx1, idx2, idx3, idx4, idx5)
    rows = (rows0, rows1, rows2, rows3)
    semg = (sg0, sg1, sg2, sg3)
    semsc = (ss0, ss1, ss2, ss3)
    semi = (si0, si1, si2, si3, si4, si5)

    one16 = jnp.full((16,), 1.0, dtype=jnp.float32)
    zero16 = jnp.zeros((16,), dtype=jnp.float32)
    for i in range(K // 16):
        ones_v[pl.ds(i * 16, 16)] = one16
    for i in range(DCH // 16):
        deg_io[pl.ds(i * 16, 16)] = zero16

    r0 = s * RPT
    pltpu.sync_copy(nfeat_hbm.at[pl.ds(r0, RPT)], acc_sh.at[pl.ds(r0, RPT)])
    for r in range(RPT // DCH):
        r0 = s * RPT + r * DCH
        pltpu.sync_copy(deg_io, deg_sh.at[pl.ds(r0, DCH)])
    plsc.subcore_barrier()

    def idx_issue(t, q):
        pltpu.async_copy(edge_hbm.at[cbase + t], idxb[q], semi[q])

    def idx_drain(q):
        pltpu.make_async_copy(edge_hbm.at[cbase], idxb[q], semi[q]).wait()

    def gather_issue(t_q, t_r):
        pltpu.async_copy(nfeat_hbm.at[idxb[t_q].at[0]], rows[t_r],
                         semg[t_r])

    def gather_wait(t_r):
        pltpu.make_async_copy(nfeat_hbm.at[idxb[0].at[0]], rows[t_r],
                              semg[t_r]).wait()

    def scat_fire(t_q, t_r):
        pltpu.async_copy(rows[t_r], acc_sh.at[idxb[t_q].at[1]], semsc[t_r],
                         add=True)
        pltpu.async_copy(ones_v, deg_sh.at[idxb[t_q].at[1]], semsc[t_r],
                         add=True)

    def scat_drain(t_q, t_r):
        pltpu.make_async_copy(rows[t_r], acc_sh.at[idxb[t_q].at[1]],
                              semsc[t_r]).wait()
        pltpu.make_async_copy(ones_v, deg_sh.at[idxb[t_q].at[1]],
                              semsc[t_r]).wait()

    def step(t, tm, drain_sc, issue_idx, drain_idx, issue_g):
        rq, rr = tm % NQ, tm % NR
        if drain_sc:
            scat_drain((tm - 2) % NQ, (tm - 2) % NR)
        if issue_idx:
            idx_issue(t + 4, (tm + 4) % NQ)
        gather_wait(rr)
        scat_fire(rq, rr)
        if drain_idx:
            idx_drain((tm + 2) % NQ)
        if issue_g:
            gather_issue((tm + 2) % NQ, (tm + 2) % NR)

    pltpu.sync_copy(edge_hbm.at[cbase + 0], idx0)
    pltpu.sync_copy(edge_hbm.at[cbase + 1], idx1)
    idx_issue(2, 2)
    idx_issue(3, 3)
    gather_issue(0, 0)
    gather_issue(1, 1)
    step(0, 0, False, True, True, True)
    step(1, 1, False, True, True, True)

    def body(h, carry):
        tb = T0 + h * BODY
        for j in range(BODY):
            step(tb + j, T0 + j, True, True, True, True)
        return carry

    lax.fori_loop(0, NLOOP, body, 0)

    for t in range(TTAIL, GRW):
        step(t, t, True, t + 4 < GRW, t + 2 < GRW, t + 2 < GRW)
    scat_drain((GRW - 2) % NQ, (GRW - 2) % NR)
    scat_drain((GRW - 1) % NQ, (GRW - 1) % NR)
    plsc.subcore_barrier()

    r0 = s * RPT
    pltpu.sync_copy(acc_sh.at[pl.ds(r0, RPT)], acc_out.at[c].at[pl.ds(r0, RPT)])
    pltpu.sync_copy(deg_sh.at[pl.ds(r0, RPT)], deg_out.at[c].at[pl.ds(r0, RPT)])


_sc_scatter = pl.kernel(
    _sc_scatter_body,
    out_type=[
        jax.ShapeDtypeStruct((NC, NP, D), jnp.float32),
        jax.ShapeDtypeStruct((NC, NP), jnp.float32),
    ],
    mesh=plsc.VectorSubcoreMesh(core_axis_name="c", subcore_axis_name="s",
                                num_cores=NC, num_subcores=NS),
    scratch_types=(
        [pltpu.VMEM((2, K), jnp.int32) for _ in range(NQ)]
        + [pltpu.VMEM((K, D), jnp.float32) for _ in range(NR)]
        + [
            pltpu.VMEM((DCH,), jnp.float32),
            pltpu.VMEM((K,), jnp.float32),
        ]
        + [pltpu.SemaphoreType.DMA for _ in range(NR + NR + NQ)]
        + [
            pltpu.VMEM_SHARED((NP, D), jnp.float32),
            pltpu.VMEM_SHARED((NP,), jnp.float32),
        ]
    ),
)


def _tc_combine_body(acc_ref, deg_ref, wt_ref, b_ref, out_ref):
    a = acc_ref[0] + acc_ref[1]
    d = deg_ref[0] + deg_ref[1] + 1.0
    y = jnp.dot(a, wt_ref[...], preferred_element_type=jnp.float32)
    out_ref[...] = (y + b_ref[...]) / d + b_ref[...]


def _tc_combine(acc, deg, wt, b):
    blk = 1000
    grid = (N_NODES // blk,)
    return pl.pallas_call(
        _tc_combine_body,
        grid=grid,
        in_specs=[
            pl.BlockSpec((NC, blk, D), lambda i: (0, i, 0)),
            pl.BlockSpec((NC, blk, 1), lambda i: (0, i, 0)),
            pl.BlockSpec((D, D), lambda i: (0, 0)),
            pl.BlockSpec((1, D), lambda i: (0, 0)),
        ],
        out_specs=pl.BlockSpec((blk, D), lambda i: (i, 0)),
        out_shape=jax.ShapeDtypeStruct((N_NODES, D), jnp.float32),
    )(acc, deg, wt, b)


def kernel(nfeat, edge_index, W_neigh, b_neigh):
    src = edge_index[0].astype(jnp.int32)
    dst = edge_index[1].astype(jnp.int32)
    npad = EP - N_EDGES
    src2 = jnp.concatenate([src, jnp.full((npad,), N_NODES, jnp.int32)])
    dst2 = jnp.concatenate([dst, jnp.full((npad,), N_NODES, jnp.int32)])
    edges = jnp.stack([src2.reshape(-1, K), dst2.reshape(-1, K)], axis=1)
    nfeat_p = jnp.pad(nfeat, ((0, NP - N_NODES), (0, 0)))
    acc, degf = _sc_scatter(nfeat_p, edges)
    return _tc_combine(acc, degf[:, :, None], W_neigh.T, b_neigh[None, :])


# speedup vs baseline: 3.4477x; 1.1360x over previous
import jax
import jax.numpy as jnp
from jax import lax
from jax.experimental import pallas as pl
from jax.experimental.pallas import tpu as pltpu
from jax.experimental.pallas import tpu_sc as plsc

N_NODES = 10000
N_EDGES = 320000
D = 128

NC = 2
NS = 16
NW = NC * NS
K = 80
GRW = -(-N_EDGES // (NW * K))
EP = NW * GRW * K
NP = 10240
RPT = NP // NS
RCH = 80
DCH = 128
NR = 4
NQ = 6
BODY = 12
T0 = 2
NLOOP = (GRW - T0 - 5) // BODY
TTAIL = T0 + NLOOP * BODY


def _sc_scatter_body(nfeat_hbm, edge_hbm,
                     acc_out, deg_out,
                     idx0, idx1, idx2, idx3, idx4, idx5,
                     rows0, rows1, rows2, rows3,
                     deg_io, ones_v,
                     sg0, sg1, sg2, sg3, ss0, ss1, ss2, ss3,
                     si0, si1, si2, si3, si4, si5,
                     acc_sh, deg_sh):
    c = lax.axis_index("c")
    s = lax.axis_index("s")
    wid = c * NS + s
    cbase = wid * GRW

    idxb = (idx0, idx1, idx2, idx3, idx4, idx5)
    rows = (rows0, rows1, rows2, rows3)
    semg = (sg0, sg1, sg2, sg3)
    semsc = (ss0, ss1, ss2, ss3)
    semi = (si0, si1, si2, si3, si4, si5)

    one16 = jnp.full((16,), 1.0, dtype=jnp.float32)
    zero16 = jnp.zeros((16,), dtype=jnp.float32)
    for i in range(K // 16):
        ones_v[pl.ds(i * 16, 16)] = one16
    for i in range(DCH // 16):
        deg_io[pl.ds(i * 16, 16)] = zero16

    r0 = s * RPT
    pltpu.sync_copy(nfeat_hbm.at[pl.ds(r0, RPT)], acc_sh.at[pl.ds(r0, RPT)])
    for r in range(RPT // DCH):
        r0 = s * RPT + r * DCH
        pltpu.sync_copy(deg_io, deg_sh.at[pl.ds(r0, DCH)])
    plsc.subcore_barrier()

    def idx_issue(t, q):
        pltpu.async_copy(edge_hbm.at[cbase + t], idxb[q], semi[q])

    def idx_drain(q):
        pltpu.make_async_copy(edge_hbm.at[cbase], idxb[q], semi[q]).wait()

    def gather_issue(t_q, t_r):
        pltpu.async_copy(nfeat_hbm.at[idxb[t_q].at[0]], rows[t_r],
                         semg[t_r])

    def gather_wait(t_r):
        pltpu.make_async_copy(nfeat_hbm.at[idxb[0].at[0]], rows[t_r],
                              semg[t_r]).wait()

    def scat_fire(t_q, t_r):
        pltpu.async_copy(rows[t_r], acc_sh.at[idxb[t_q].at[1]], semsc[t_r],
                         add=True)
        pltpu.async_copy(ones_v, deg_sh.at[idxb[t_q].at[1]], semsc[t_r],
                         add=True)

    def scat_drain(t_q, t_r):
        pltpu.make_async_copy(rows[t_r], acc_sh.at[idxb[t_q].at[1]],
                              semsc[t_r]).wait()
        pltpu.make_async_copy(ones_v, deg_sh.at[idxb[t_q].at[1]],
                              semsc[t_r]).wait()

    def step(t, tm, drain_sc, issue_idx, drain_idx, issue_g):
        rq, rr = tm % NQ, tm % NR
        if drain_sc:
            scat_drain((tm - 1) % NQ, (tm - 1) % NR)
        if issue_idx:
            idx_issue(t + 5, (tm + 5) % NQ)
        gather_wait(rr)
        scat_fire(rq, rr)
        if drain_idx:
            idx_drain((tm + 3) % NQ)
        if issue_g:
            gather_issue((tm + 3) % NQ, (tm + 3) % NR)

    pltpu.sync_copy(edge_hbm.at[cbase + 0], idx0)
    pltpu.sync_copy(edge_hbm.at[cbase + 1], idx1)
    idx_issue(2, 2)
    idx_issue(3, 3)
    idx_issue(4, 4)
    gather_issue(0, 0)
    gather_issue(1, 1)
    idx_drain(2)
    gather_issue(2, 2)
    step(0, 0, False, True, True, True)
    step(1, 1, True, True, True, True)

    def body(h, carry):
        tb = T0 + h * BODY
        for j in range(BODY):
            step(tb + j, T0 + j, True, True, True, True)
        return carry

    lax.fori_loop(0, NLOOP, body, 0)

    for t in range(TTAIL, GRW):
        step(t, t, True, t + 5 < GRW, t + 3 < GRW, t + 3 < GRW)
    scat_drain((GRW - 1) % NQ, (GRW - 1) % NR)
    plsc.subcore_barrier()

    r0 = s * RPT
    pltpu.sync_copy(acc_sh.at[pl.ds(r0, RPT)], acc_out.at[c].at[pl.ds(r0, RPT)])
    pltpu.sync_copy(deg_sh.at[pl.ds(r0, RPT)], deg_out.at[c].at[pl.ds(r0, RPT)])


_sc_scatter = pl.kernel(
    _sc_scatter_body,
    out_type=[
        jax.ShapeDtypeStruct((NC, NP, D), jnp.float32),
        jax.ShapeDtypeStruct((NC, NP), jnp.float32),
    ],
    mesh=plsc.VectorSubcoreMesh(core_axis_name="c", subcore_axis_name="s",
                                num_cores=NC, num_subcores=NS),
    scratch_types=(
        [pltpu.VMEM((2, K), jnp.int32) for _ in range(NQ)]
        + [pltpu.VMEM((K, D), jnp.float32) for _ in range(NR)]
        + [
            pltpu.VMEM((DCH,), jnp.float32),
            pltpu.VMEM((K,), jnp.float32),
        ]
        + [pltpu.SemaphoreType.DMA for _ in range(NR + NR + NQ)]
        + [
            pltpu.VMEM_SHARED((NP, D), jnp.float32),
            pltpu.VMEM_SHARED((NP,), jnp.float32),
        ]
    ),
)


def _tc_combine_body(acc_ref, deg_ref, wt_ref, b_ref, out_ref):
    a = acc_ref[0] + acc_ref[1]
    d = deg_ref[0] + deg_ref[1] + 1.0
    y = jnp.dot(a, wt_ref[...], preferred_element_type=jnp.float32)
    out_ref[...] = (y + b_ref[...]) / d + b_ref[...]


def _tc_combine(acc, deg, wt, b):
    blk = 1000
    grid = (N_NODES // blk,)
    return pl.pallas_call(
        _tc_combine_body,
        grid=grid,
        in_specs=[
            pl.BlockSpec((NC, blk, D), lambda i: (0, i, 0)),
            pl.BlockSpec((NC, blk, 1), lambda i: (0, i, 0)),
            pl.BlockSpec((D, D), lambda i: (0, 0)),
            pl.BlockSpec((1, D), lambda i: (0, 0)),
        ],
        out_specs=pl.BlockSpec((blk, D), lambda i: (i, 0)),
        out_shape=jax.ShapeDtypeStruct((N_NODES, D), jnp.float32),
    )(acc, deg, wt, b)


def kernel(nfeat, edge_index, W_neigh, b_neigh):
    src = edge_index[0].astype(jnp.int32)
    dst = edge_index[1].astype(jnp.int32)
    npad = EP - N_EDGES
    src2 = jnp.concatenate([src, jnp.full((npad,), N_NODES, jnp.int32)])
    dst2 = jnp.concatenate([dst, jnp.full((npad,), N_NODES, jnp.int32)])
    edges = jnp.stack([src2.reshape(-1, K), dst2.reshape(-1, K)], axis=1)
    nfeat_p = jnp.pad(nfeat, ((0, NP - N_NODES), (0, 0)))
    acc, degf = _sc_scatter(nfeat_p, edges)
    return _tc_combine(acc, degf[:, :, None], W_neigh.T, b_neigh[None, :])
